# baseline (speedup 1.0000x reference)
import functools

import jax
import jax.numpy as jnp
from jax import lax
from jax.experimental import pallas as pl
from jax.experimental.pallas import tpu as pltpu
from jax.experimental.pallas import tpu_sc as plsc

B, Q, L = 16, 5000, 256
QBLK = 512
QPAD = 5120
NLANE = 16
BLKSZ = QPAD // NLANE
NVREG = QPAD // NLANE


def _scores_body(pm_ref, logits_ref, out_ref):
    pos = pm_ref[...] > 1e-6
    logits = logits_ref[...]
    sig = jax.nn.sigmoid(logits)
    masked = jnp.where(pos[:, None, :], sig, 0.0)
    score = jnp.max(masked, axis=-1)
    qid = pl.program_id(0) * QBLK + lax.broadcasted_iota(jnp.int32, (B, QBLK), 1)
    out_ref[...] = jnp.where(qid < Q, score, 0.0)


def _scores(pred_logits, positive_map):
    return pl.pallas_call(
        _scores_body,
        grid=(QPAD // QBLK,),
        in_specs=[
            pl.BlockSpec((B, L), lambda q: (0, 0)),
            pl.BlockSpec((B, QBLK, L), lambda q: (0, q, 0)),
        ],
        out_specs=pl.BlockSpec((B, QBLK), lambda q: (0, q)),
        out_shape=jax.ShapeDtypeStruct((B, QPAD), jnp.float32),
    )(positive_map, pred_logits)


def _sort_gather_body(scores_hbm, boxes_hbm, tsw_hbm, tsh_hbm,
                      out_hbm,
                      kA, iA, kB, iB, hist, bx, ov, tsw, tsh):
    wid = lax.axis_index("s") * 2 + lax.axis_index("c")

    @pl.when(wid < B)
    def _():
        lanes = lax.iota(jnp.int32, NLANE)
        ones = jnp.ones((NLANE,), jnp.int32)

        pltpu.sync_copy(scores_hbm.at[wid], kA)
        pltpu.sync_copy(boxes_hbm.at[wid], bx)
        pltpu.sync_copy(tsw_hbm.at[wid], tsw)
        pltpu.sync_copy(tsh_hbm.at[wid], tsh)
        img_w = tsw[pl.ds(0, NLANE)]
        img_h = tsh[pl.ds(0, NLANE)]

        @plsc.parallel_loop(0, NVREG, unroll=8)
        def init_idx(i):
            iA[pl.ds(i * NLANE, NLANE)] = lanes + i * NLANE

        gblock = lanes * BLKSZ
        for p in range(4):
            src_k, src_i = (kA, iA) if p % 2 == 0 else (kB, iB)
            dst_k, dst_i = (kB, iB) if p % 2 == 0 else (kA, iA)
            shift = 8 * p

            @plsc.parallel_loop(0, 256, unroll=8)
            def zero_hist(i):
                hist[pl.ds(i * NLANE, NLANE)] = jnp.zeros((NLANE,), jnp.int32)

            @plsc.parallel_loop(0, BLKSZ, unroll=4)
            def histo(t):
                kv = plsc.load_gather(src_k, [gblock + t])
                d = 255 - ((plsc.bitcast(kv, jnp.int32) >> shift) & 0xFF)
                plsc.addupdate_scatter(hist, [d * NLANE + lanes], ones)

            def scan(i, carry):
                v = hist[pl.ds(i * NLANE, NLANE)]
                cum = plsc.cumsum(v)
                hist[pl.ds(i * NLANE, NLANE)] = cum - v + carry
                return carry + cum[NLANE - 1]
            lax.fori_loop(0, 256, scan, jnp.int32(0))

            def permute(t, _):
                gidx = gblock + t
                kv = plsc.load_gather(src_k, [gidx])
                iv = plsc.load_gather(src_i, [gidx])
                d = 255 - ((plsc.bitcast(kv, jnp.int32) >> shift) & 0xFF)
                slot = d * NLANE + lanes
                pos = plsc.load_gather(hist, [slot])
                plsc.addupdate_scatter(hist, [slot], ones)
                plsc.store_scatter(dst_k, [pos], kv)
                plsc.store_scatter(dst_i, [pos], iv)
                return 0
            lax.fori_loop(0, BLKSZ, permute, 0)

        @plsc.parallel_loop(0, NVREG, unroll=4)
        def gather(i):
            si = iA[pl.ds(i * NLANE, NLANE)] * 4
            cx = plsc.load_gather(bx, [si])
            cy = plsc.load_gather(bx, [si + 1])
            w = plsc.load_gather(bx, [si + 2])
            h = plsc.load_gather(bx, [si + 3])
            rb = (lanes + i * NLANE) * 4
            plsc.store_scatter(ov, [rb], (cx - 0.5 * w) * img_w)
            plsc.store_scatter(ov, [rb + 1], (cy - 0.5 * h) * img_h)
            plsc.store_scatter(ov, [rb + 2], (cx + 0.5 * w) * img_w)
            plsc.store_scatter(ov, [rb + 3], (cy + 0.5 * h) * img_h)

        pltpu.sync_copy(ov, out_hbm.at[wid])


def _sort_gather(scores, pred_boxes, target_sizes):
    mesh = plsc.VectorSubcoreMesh(core_axis_name="c", subcore_axis_name="s")
    kern = pl.kernel(
        _sort_gather_body,
        mesh=mesh,
        compiler_params=pltpu.CompilerParams(needs_layout_passes=False),
        out_type=jax.ShapeDtypeStruct((B, QPAD * 4), jnp.float32),
        scratch_types=[
            pltpu.VMEM((QPAD,), jnp.float32),
            pltpu.VMEM((QPAD,), jnp.int32),
            pltpu.VMEM((QPAD,), jnp.float32),
            pltpu.VMEM((QPAD,), jnp.int32),
            pltpu.VMEM((256 * NLANE,), jnp.int32),
            pltpu.VMEM((QPAD * 4,), jnp.float32),
            pltpu.VMEM((QPAD * 4,), jnp.float32),
            pltpu.VMEM((128,), jnp.float32),
            pltpu.VMEM((128,), jnp.float32),
        ],
    )
    tsw = jnp.broadcast_to(target_sizes[:, 1:2], (B, 128))
    tsh = jnp.broadcast_to(target_sizes[:, 0:1], (B, 128))
    boxes_flat = jnp.pad(pred_boxes.reshape(B, Q * 4), ((0, 0), (0, (QPAD - Q) * 4)))
    return kern(scores, boxes_flat, tsw, tsh)


def kernel(pred_logits, pred_boxes, target_sizes, positive_map, items_per_batch_element):
    scores = _scores(pred_logits, positive_map)
    out_flat = _sort_gather(scores, pred_boxes, target_sizes)
    return out_flat.reshape(B, QPAD, 4)[:, :Q, :]


# speedup vs baseline: 1.1849x; 1.1849x over previous
import functools

import jax
import jax.numpy as jnp
from jax import lax
from jax.experimental import pallas as pl
from jax.experimental.pallas import tpu as pltpu
from jax.experimental.pallas import tpu_sc as plsc

B, Q, L = 16, 5000, 256
QBLK = 512
QPAD = 5120
NLANE = 16
BLKSZ = QPAD // NLANE
NVREG = QPAD // NLANE


def _scores_body(pm_ref, logits_ref, out_ref):
    pos = pm_ref[...] > 1e-6
    logits = logits_ref[...]
    sig = jax.nn.sigmoid(logits)
    masked = jnp.where(pos[:, None, :], sig, 0.0)
    score = jnp.max(masked, axis=-1)
    qid = pl.program_id(0) * QBLK + lax.broadcasted_iota(jnp.int32, (B, QBLK), 1)
    out_ref[...] = jnp.where(qid < Q, score, 0.0)


def _scores(pred_logits, positive_map):
    return pl.pallas_call(
        _scores_body,
        grid=(QPAD // QBLK,),
        in_specs=[
            pl.BlockSpec((B, L), lambda q: (0, 0)),
            pl.BlockSpec((B, QBLK, L), lambda q: (0, q, 0)),
        ],
        out_specs=pl.BlockSpec((B, QBLK), lambda q: (0, q)),
        out_shape=jax.ShapeDtypeStruct((B, QPAD), jnp.float32),
    )(positive_map, pred_logits)


def _sort_gather_body(scores_hbm, boxes_hbm, tsw_hbm, tsh_hbm,
                      out_hbm,
                      kA, iA, kB, iB, hist, bx, ov, tsw, tsh):
    wid = lax.axis_index("s") * 2 + lax.axis_index("c")

    @pl.when(wid < B)
    def _():
        lanes = lax.iota(jnp.int32, NLANE)
        ones = jnp.ones((NLANE,), jnp.int32)

        pltpu.sync_copy(scores_hbm.at[wid], kA)
        pltpu.sync_copy(boxes_hbm.at[wid], bx)
        pltpu.sync_copy(tsw_hbm.at[wid], tsw)
        pltpu.sync_copy(tsh_hbm.at[wid], tsh)
        img_w = tsw[pl.ds(0, NLANE)]
        img_h = tsh[pl.ds(0, NLANE)]

        @plsc.parallel_loop(0, NVREG, unroll=8)
        def init_idx(i):
            iA[pl.ds(i * NLANE, NLANE)] = lanes + i * NLANE

        gblock = lanes * BLKSZ
        for p in range(4):
            src_k, src_i = (kA, iA) if p % 2 == 0 else (kB, iB)
            dst_k, dst_i = (kB, iB) if p % 2 == 0 else (kA, iA)
            shift = 8 * p

            @plsc.parallel_loop(0, 256, unroll=8)
            def zero_hist(i):
                hist[pl.ds(i * NLANE, NLANE)] = jnp.zeros((NLANE,), jnp.int32)

            @plsc.parallel_loop(0, BLKSZ, unroll=4)
            def histo(t):
                kv = plsc.load_gather(src_k, [gblock + t])
                d = 255 - ((plsc.bitcast(kv, jnp.int32) >> shift) & 0xFF)
                plsc.addupdate_scatter(hist, [d * NLANE + lanes], ones)

            def scan(i, carry):
                v = hist[pl.ds(i * NLANE, NLANE)]
                cum = plsc.cumsum(v)
                hist[pl.ds(i * NLANE, NLANE)] = cum - v + carry
                return carry + cum[NLANE - 1]
            lax.fori_loop(0, 256, scan, jnp.int32(0))

            def permute(t, _):
                gidx = gblock + t
                kv = plsc.load_gather(src_k, [gidx])
                iv = plsc.load_gather(src_i, [gidx])
                d = 255 - ((plsc.bitcast(kv, jnp.int32) >> shift) & 0xFF)
                slot = d * NLANE + lanes
                pos = plsc.load_gather(hist, [slot])
                plsc.addupdate_scatter(hist, [slot], ones)
                plsc.store_scatter(dst_k, [pos], kv)
                plsc.store_scatter(dst_i, [pos], iv)
                return 0
            lax.fori_loop(0, BLKSZ, permute, 0)

        @plsc.parallel_loop(0, NVREG, unroll=4)
        def gather(i):
            si = iA[pl.ds(i * NLANE, NLANE)] * 4
            cx = plsc.load_gather(bx, [si])
            cy = plsc.load_gather(bx, [si + 1])
            w = plsc.load_gather(bx, [si + 2])
            h = plsc.load_gather(bx, [si + 3])
            sl = pl.ds(i * NLANE, NLANE)
            ov[0, sl] = (cx - 0.5 * w) * img_w
            ov[1, sl] = (cy - 0.5 * h) * img_h
            ov[2, sl] = (cx + 0.5 * w) * img_w
            ov[3, sl] = (cy + 0.5 * h) * img_h

        pltpu.sync_copy(ov, out_hbm.at[wid])


def _sort_gather(scores, pred_boxes, target_sizes):
    mesh = plsc.VectorSubcoreMesh(core_axis_name="c", subcore_axis_name="s")
    kern = pl.kernel(
        _sort_gather_body,
        mesh=mesh,
        compiler_params=pltpu.CompilerParams(needs_layout_passes=False),
        out_type=jax.ShapeDtypeStruct((B, 4, QPAD), jnp.float32),
        scratch_types=[
            pltpu.VMEM((QPAD,), jnp.float32),
            pltpu.VMEM((QPAD,), jnp.int32),
            pltpu.VMEM((QPAD,), jnp.float32),
            pltpu.VMEM((QPAD,), jnp.int32),
            pltpu.VMEM((256 * NLANE,), jnp.int32),
            pltpu.VMEM((QPAD * 4,), jnp.float32),
            pltpu.VMEM((4, QPAD), jnp.float32),
            pltpu.VMEM((128,), jnp.float32),
            pltpu.VMEM((128,), jnp.float32),
        ],
    )
    tsw = jnp.broadcast_to(target_sizes[:, 1:2], (B, 128))
    tsh = jnp.broadcast_to(target_sizes[:, 0:1], (B, 128))
    boxes_flat = jnp.pad(pred_boxes.reshape(B, Q * 4), ((0, 0), (0, (QPAD - Q) * 4)))
    return kern(scores, boxes_flat, tsw, tsh)


def kernel(pred_logits, pred_boxes, target_sizes, positive_map, items_per_batch_element):
    scores = _scores(pred_logits, positive_map)
    out_t = _sort_gather(scores, pred_boxes, target_sizes)
    return out_t.transpose(0, 2, 1)[:, :Q, :]


# speedup vs baseline: 1.2565x; 1.0605x over previous
import functools

import jax
import jax.numpy as jnp
from jax import lax
from jax.experimental import pallas as pl
from jax.experimental.pallas import tpu as pltpu
from jax.experimental.pallas import tpu_sc as plsc

B, Q, L = 16, 5000, 256
QBLK = 1024
QPAD = 5120
NLANE = 16
BLKSZ = QPAD // NLANE
NVREG = QPAD // NLANE


def _scores_body(pm_ref, logits_ref, out_ref):
    pos = pm_ref[...] > 1e-6
    logits = logits_ref[...]
    sig = jax.nn.sigmoid(logits)
    masked = jnp.where(pos[:, None, :], sig, 0.0)
    score = jnp.max(masked, axis=-1)
    qid = pl.program_id(0) * QBLK + lax.broadcasted_iota(jnp.int32, (B, QBLK), 1)
    out_ref[...] = jnp.where(qid < Q, score, 0.0)


def _scores(pred_logits, positive_map):
    return pl.pallas_call(
        _scores_body,
        grid=(QPAD // QBLK,),
        in_specs=[
            pl.BlockSpec((B, L), lambda q: (0, 0)),
            pl.BlockSpec((B, QBLK, L), lambda q: (0, q, 0)),
        ],
        out_specs=pl.BlockSpec((B, QBLK), lambda q: (0, q)),
        out_shape=jax.ShapeDtypeStruct((B, QPAD), jnp.float32),
    )(positive_map, pred_logits)


def _sort_gather_body(scores_hbm, boxes_hbm, tsw_hbm, tsh_hbm,
                      out_hbm,
                      kA, iA, kB, iB, hist, bx, ov, tsw, tsh):
    wid = lax.axis_index("s") * 2 + lax.axis_index("c")

    @pl.when(wid < B)
    def _():
        lanes = lax.iota(jnp.int32, NLANE)
        ones = jnp.ones((NLANE,), jnp.int32)

        pltpu.sync_copy(scores_hbm.at[wid], kA)
        pltpu.sync_copy(boxes_hbm.at[wid], bx)
        pltpu.sync_copy(tsw_hbm.at[wid], tsw)
        pltpu.sync_copy(tsh_hbm.at[wid], tsh)
        img_w = tsw[pl.ds(0, NLANE)]
        img_h = tsh[pl.ds(0, NLANE)]

        @plsc.parallel_loop(0, NVREG, unroll=8)
        def init_idx(i):
            iA[pl.ds(i * NLANE, NLANE)] = lanes + i * NLANE

        gblock = lanes * BLKSZ
        for p in range(4):
            src_k, src_i = (kA, iA) if p % 2 == 0 else (kB, iB)
            dst_k, dst_i = (kB, iB) if p % 2 == 0 else (kA, iA)
            shift = 8 * p

            @plsc.parallel_loop(0, 256, unroll=8)
            def zero_hist(i):
                hist[pl.ds(i * NLANE, NLANE)] = jnp.zeros((NLANE,), jnp.int32)

            @plsc.parallel_loop(0, BLKSZ, unroll=4)
            def histo(t):
                kv = plsc.load_gather(src_k, [gblock + t])
                d = 255 - ((plsc.bitcast(kv, jnp.int32) >> shift) & 0xFF)
                plsc.addupdate_scatter(hist, [d * NLANE + lanes], ones)

            def scan(ic, carry):
                for u in range(4):
                    sl = pl.ds((ic * 4 + u) * NLANE, NLANE)
                    v = hist[sl]
                    cum = plsc.cumsum(v)
                    hist[sl] = cum - v + carry
                    carry = carry + cum[NLANE - 1]
                return carry
            lax.fori_loop(0, 256 // 4, scan, jnp.int32(0))

            def permute(tc, _):
                for u in range(8):
                    gidx = gblock + (tc * 8 + u)
                    kv = plsc.load_gather(src_k, [gidx])
                    iv = plsc.load_gather(src_i, [gidx])
                    d = 255 - ((plsc.bitcast(kv, jnp.int32) >> shift) & 0xFF)
                    slot = d * NLANE + lanes
                    pos = plsc.load_gather(hist, [slot])
                    plsc.addupdate_scatter(hist, [slot], ones)
                    plsc.store_scatter(dst_k, [pos], kv)
                    plsc.store_scatter(dst_i, [pos], iv)
                return 0
            lax.fori_loop(0, BLKSZ // 8, permute, 0)

        @plsc.parallel_loop(0, NVREG, unroll=4)
        def gather(i):
            si = iA[pl.ds(i * NLANE, NLANE)] * 4
            cx = plsc.load_gather(bx, [si])
            cy = plsc.load_gather(bx, [si + 1])
            w = plsc.load_gather(bx, [si + 2])
            h = plsc.load_gather(bx, [si + 3])
            sl = pl.ds(i * NLANE, NLANE)
            ov[0, sl] = (cx - 0.5 * w) * img_w
            ov[1, sl] = (cy - 0.5 * h) * img_h
            ov[2, sl] = (cx + 0.5 * w) * img_w
            ov[3, sl] = (cy + 0.5 * h) * img_h

        pltpu.sync_copy(ov, out_hbm.at[wid])


def _sort_gather(scores, pred_boxes, target_sizes):
    mesh = plsc.VectorSubcoreMesh(core_axis_name="c", subcore_axis_name="s")
    kern = pl.kernel(
        _sort_gather_body,
        mesh=mesh,
        compiler_params=pltpu.CompilerParams(needs_layout_passes=False),
        out_type=jax.ShapeDtypeStruct((B, 4, QPAD), jnp.float32),
        scratch_types=[
            pltpu.VMEM((QPAD,), jnp.float32),
            pltpu.VMEM((QPAD,), jnp.int32),
            pltpu.VMEM((QPAD,), jnp.float32),
            pltpu.VMEM((QPAD,), jnp.int32),
            pltpu.VMEM((256 * NLANE,), jnp.int32),
            pltpu.VMEM((QPAD * 4,), jnp.float32),
            pltpu.VMEM((4, QPAD), jnp.float32),
            pltpu.VMEM((128,), jnp.float32),
            pltpu.VMEM((128,), jnp.float32),
        ],
    )
    tsw = jnp.broadcast_to(target_sizes[:, 1:2], (B, 128))
    tsh = jnp.broadcast_to(target_sizes[:, 0:1], (B, 128))
    boxes_flat = jnp.pad(pred_boxes.reshape(B, Q * 4), ((0, 0), (0, (QPAD - Q) * 4)))
    return kern(scores, boxes_flat, tsw, tsh)


def kernel(pred_logits, pred_boxes, target_sizes, positive_map, items_per_batch_element):
    scores = _scores(pred_logits, positive_map)
    out_t = _sort_gather(scores, pred_boxes, target_sizes)
    return out_t.transpose(0, 2, 1)[:, :Q, :]


# speedup vs baseline: 1.2676x; 1.0088x over previous
import functools

import jax
import jax.numpy as jnp
from jax import lax
from jax.experimental import pallas as pl
from jax.experimental.pallas import tpu as pltpu
from jax.experimental.pallas import tpu_sc as plsc

B, Q, L = 16, 5000, 256
QBLK = 1024
QPAD = 5120
NLANE = 16
BLKSZ = QPAD // NLANE
NVREG = QPAD // NLANE


def _scores_body(pm_ref, logits_ref, out_ref):
    pos = pm_ref[...] > 1e-6
    logits = logits_ref[...]
    sig = jax.nn.sigmoid(logits)
    masked = jnp.where(pos[:, None, :], sig, 0.0)
    score = jnp.max(masked, axis=-1)
    qid = pl.program_id(0) * QBLK + lax.broadcasted_iota(jnp.int32, (B, QBLK), 1)
    out_ref[...] = jnp.where(qid < Q, score, 0.0)


def _scores(pred_logits, positive_map):
    return pl.pallas_call(
        _scores_body,
        grid=(QPAD // QBLK,),
        in_specs=[
            pl.BlockSpec((B, L), lambda q: (0, 0)),
            pl.BlockSpec((B, QBLK, L), lambda q: (0, q, 0)),
        ],
        out_specs=pl.BlockSpec((B, QBLK), lambda q: (0, q)),
        out_shape=jax.ShapeDtypeStruct((B, QPAD), jnp.float32),
    )(positive_map, pred_logits)


def _sort_gather_body(scores_hbm, boxes_hbm, tsw_hbm, tsh_hbm,
                      out_hbm,
                      kA, iA, kB, iB, hist, bx, ov, tsw, tsh):
    wid = lax.axis_index("s") * 2 + lax.axis_index("c")

    @pl.when(wid < B)
    def _():
        lanes = lax.iota(jnp.int32, NLANE)
        ones = jnp.ones((NLANE,), jnp.int32)

        pltpu.sync_copy(scores_hbm.at[wid], kA)
        pltpu.sync_copy(boxes_hbm.at[wid], bx)
        pltpu.sync_copy(tsw_hbm.at[wid], tsw)
        pltpu.sync_copy(tsh_hbm.at[wid], tsh)
        img_w = tsw[pl.ds(0, NLANE)]
        img_h = tsh[pl.ds(0, NLANE)]

        @plsc.parallel_loop(0, NVREG, unroll=8)
        def init_idx(i):
            iA[pl.ds(i * NLANE, NLANE)] = lanes + i * NLANE

        gblock = lanes * BLKSZ
        for p in range(4):
            src_k, src_i = (kA, iA) if p % 2 == 0 else (kB, iB)
            dst_k, dst_i = (kB, iB) if p % 2 == 0 else (kA, iA)
            shift = 8 * p

            @plsc.parallel_loop(0, 256, unroll=8)
            def zero_hist(i):
                hist[pl.ds(i * NLANE, NLANE)] = jnp.zeros((NLANE,), jnp.int32)

            @plsc.parallel_loop(0, BLKSZ, unroll=8)
            def histo(t):
                kv = plsc.load_gather(src_k, [gblock + t])
                d = 255 - ((plsc.bitcast(kv, jnp.int32) >> shift) & 0xFF)
                plsc.addupdate_scatter(hist, [d * NLANE + lanes], ones)

            def scan(ic, carry):
                for u in range(8):
                    sl = pl.ds((ic * 8 + u) * NLANE, NLANE)
                    v = hist[sl]
                    cum = plsc.cumsum(v)
                    hist[sl] = cum - v + carry
                    carry = carry + cum[NLANE - 1]
                return carry
            lax.fori_loop(0, 256 // 8, scan, jnp.int32(0))

            def permute(tc, _):
                for u in range(16):
                    gidx = gblock + (tc * 16 + u)
                    kv = plsc.load_gather(src_k, [gidx])
                    iv = plsc.load_gather(src_i, [gidx])
                    d = 255 - ((plsc.bitcast(kv, jnp.int32) >> shift) & 0xFF)
                    slot = d * NLANE + lanes
                    pos = plsc.load_gather(hist, [slot])
                    plsc.addupdate_scatter(hist, [slot], ones)
                    plsc.store_scatter(dst_k, [pos], kv)
                    plsc.store_scatter(dst_i, [pos], iv)
                return 0
            lax.fori_loop(0, BLKSZ // 16, permute, 0)

        @plsc.parallel_loop(0, NVREG, unroll=8)
        def gather(i):
            si = iA[pl.ds(i * NLANE, NLANE)] * 4
            cx = plsc.load_gather(bx, [si])
            cy = plsc.load_gather(bx, [si + 1])
            w = plsc.load_gather(bx, [si + 2])
            h = plsc.load_gather(bx, [si + 3])
            sl = pl.ds(i * NLANE, NLANE)
            ov[0, sl] = (cx - 0.5 * w) * img_w
            ov[1, sl] = (cy - 0.5 * h) * img_h
            ov[2, sl] = (cx + 0.5 * w) * img_w
            ov[3, sl] = (cy + 0.5 * h) * img_h

        pltpu.sync_copy(ov, out_hbm.at[wid])


def _sort_gather(scores, pred_boxes, target_sizes):
    mesh = plsc.VectorSubcoreMesh(core_axis_name="c", subcore_axis_name="s")
    kern = pl.kernel(
        _sort_gather_body,
        mesh=mesh,
        compiler_params=pltpu.CompilerParams(needs_layout_passes=False),
        out_type=jax.ShapeDtypeStruct((B, 4, QPAD), jnp.float32),
        scratch_types=[
            pltpu.VMEM((QPAD,), jnp.float32),
            pltpu.VMEM((QPAD,), jnp.int32),
            pltpu.VMEM((QPAD,), jnp.float32),
            pltpu.VMEM((QPAD,), jnp.int32),
            pltpu.VMEM((256 * NLANE,), jnp.int32),
            pltpu.VMEM((QPAD * 4,), jnp.float32),
            pltpu.VMEM((4, QPAD), jnp.float32),
            pltpu.VMEM((128,), jnp.float32),
            pltpu.VMEM((128,), jnp.float32),
        ],
    )
    tsw = jnp.broadcast_to(target_sizes[:, 1:2], (B, 128))
    tsh = jnp.broadcast_to(target_sizes[:, 0:1], (B, 128))
    boxes_flat = jnp.pad(pred_boxes.reshape(B, Q * 4), ((0, 0), (0, (QPAD - Q) * 4)))
    return kern(scores, boxes_flat, tsw, tsh)


def kernel(pred_logits, pred_boxes, target_sizes, positive_map, items_per_batch_element):
    scores = _scores(pred_logits, positive_map)
    out_t = _sort_gather(scores, pred_boxes, target_sizes)
    return out_t.transpose(0, 2, 1)[:, :Q, :]


# speedup vs baseline: 1.2940x; 1.0209x over previous
import functools

import jax
import jax.numpy as jnp
from jax import lax
from jax.experimental import pallas as pl
from jax.experimental.pallas import tpu as pltpu
from jax.experimental.pallas import tpu_sc as plsc

B, Q, L = 16, 5000, 256
QBLK = 1024
QPAD = 5120
NLANE = 16
BLKSZ = QPAD // NLANE
NVREG = QPAD // NLANE


def _scores_body(pm_ref, logits_ref, out_ref):
    pos = pm_ref[...] > 1e-6
    logits = logits_ref[...]
    m = jnp.max(jnp.where(pos[:, None, :], logits, -jnp.inf), axis=-1)
    score = jax.nn.sigmoid(m)
    qid = pl.program_id(0) * QBLK + lax.broadcasted_iota(jnp.int32, (B, QBLK), 1)
    out_ref[...] = jnp.where(qid < Q, score, 0.0)


def _scores(pred_logits, positive_map):
    return pl.pallas_call(
        _scores_body,
        grid=(QPAD // QBLK,),
        in_specs=[
            pl.BlockSpec((B, L), lambda q: (0, 0)),
            pl.BlockSpec((B, QBLK, L), lambda q: (0, q, 0)),
        ],
        out_specs=pl.BlockSpec((B, QBLK), lambda q: (0, q)),
        out_shape=jax.ShapeDtypeStruct((B, QPAD), jnp.float32),
    )(positive_map, pred_logits)


def _sort_gather_body(scores_hbm, boxes_hbm, tsw_hbm, tsh_hbm,
                      out_hbm,
                      kA, iA, kB, iB, hist, bx, ov, tsw, tsh):
    wid = lax.axis_index("s") * 2 + lax.axis_index("c")

    @pl.when(wid < B)
    def _():
        lanes = lax.iota(jnp.int32, NLANE)
        ones = jnp.ones((NLANE,), jnp.int32)

        pltpu.sync_copy(scores_hbm.at[wid], kA)
        pltpu.sync_copy(boxes_hbm.at[wid], bx)
        pltpu.sync_copy(tsw_hbm.at[wid], tsw)
        pltpu.sync_copy(tsh_hbm.at[wid], tsh)
        img_w = tsw[pl.ds(0, NLANE)]
        img_h = tsh[pl.ds(0, NLANE)]

        @plsc.parallel_loop(0, NVREG, unroll=8)
        def init_idx(i):
            iA[pl.ds(i * NLANE, NLANE)] = lanes + i * NLANE

        gblock = lanes * BLKSZ
        for p in range(4):
            src_k, src_i = (kA, iA) if p % 2 == 0 else (kB, iB)
            dst_k, dst_i = (kB, iB) if p % 2 == 0 else (kA, iA)
            shift = 8 * p

            @plsc.parallel_loop(0, 256, unroll=8)
            def zero_hist(i):
                hist[pl.ds(i * NLANE, NLANE)] = jnp.zeros((NLANE,), jnp.int32)

            @plsc.parallel_loop(0, BLKSZ, unroll=8)
            def histo(t):
                kv = plsc.load_gather(src_k, [gblock + t])
                d = 255 - ((plsc.bitcast(kv, jnp.int32) >> shift) & 0xFF)
                plsc.addupdate_scatter(hist, [d * NLANE + lanes], ones)

            def scan(ic, carry):
                for u in range(8):
                    sl = pl.ds((ic * 8 + u) * NLANE, NLANE)
                    v = hist[sl]
                    cum = plsc.cumsum(v)
                    hist[sl] = cum - v + carry
                    carry = carry + cum[NLANE - 1]
                return carry
            lax.fori_loop(0, 256 // 8, scan, jnp.int32(0))

            def permute(tc, _):
                for u in range(16):
                    gidx = gblock + (tc * 16 + u)
                    kv = plsc.load_gather(src_k, [gidx])
                    iv = plsc.load_gather(src_i, [gidx])
                    d = 255 - ((plsc.bitcast(kv, jnp.int32) >> shift) & 0xFF)
                    slot = d * NLANE + lanes
                    pos = plsc.load_gather(hist, [slot])
                    plsc.addupdate_scatter(hist, [slot], ones)
                    plsc.store_scatter(dst_k, [pos], kv)
                    plsc.store_scatter(dst_i, [pos], iv)
                return 0
            lax.fori_loop(0, BLKSZ // 16, permute, 0)

        @plsc.parallel_loop(0, NVREG, unroll=8)
        def gather(i):
            si = iA[pl.ds(i * NLANE, NLANE)] * 4
            cx = plsc.load_gather(bx, [si])
            cy = plsc.load_gather(bx, [si + 1])
            w = plsc.load_gather(bx, [si + 2])
            h = plsc.load_gather(bx, [si + 3])
            sl = pl.ds(i * NLANE, NLANE)
            ov[0, sl] = (cx - 0.5 * w) * img_w
            ov[1, sl] = (cy - 0.5 * h) * img_h
            ov[2, sl] = (cx + 0.5 * w) * img_w
            ov[3, sl] = (cy + 0.5 * h) * img_h

        pltpu.sync_copy(ov, out_hbm.at[wid])


def _sort_gather(scores, pred_boxes, target_sizes):
    mesh = plsc.VectorSubcoreMesh(core_axis_name="c", subcore_axis_name="s")
    kern = pl.kernel(
        _sort_gather_body,
        mesh=mesh,
        compiler_params=pltpu.CompilerParams(needs_layout_passes=False),
        out_type=jax.ShapeDtypeStruct((B, 4, QPAD), jnp.float32),
        scratch_types=[
            pltpu.VMEM((QPAD,), jnp.float32),
            pltpu.VMEM((QPAD,), jnp.int32),
            pltpu.VMEM((QPAD,), jnp.float32),
            pltpu.VMEM((QPAD,), jnp.int32),
            pltpu.VMEM((256 * NLANE,), jnp.int32),
            pltpu.VMEM((QPAD * 4,), jnp.float32),
            pltpu.VMEM((4, QPAD), jnp.float32),
            pltpu.VMEM((128,), jnp.float32),
            pltpu.VMEM((128,), jnp.float32),
        ],
    )
    tsw = jnp.broadcast_to(target_sizes[:, 1:2], (B, 128))
    tsh = jnp.broadcast_to(target_sizes[:, 0:1], (B, 128))
    boxes_flat = jnp.pad(pred_boxes.reshape(B, Q * 4), ((0, 0), (0, (QPAD - Q) * 4)))
    return kern(scores, boxes_flat, tsw, tsh)


def kernel(pred_logits, pred_boxes, target_sizes, positive_map, items_per_batch_element):
    scores = _scores(pred_logits, positive_map)
    out_t = _sort_gather(scores, pred_boxes, target_sizes)
    return out_t.transpose(0, 2, 1)[:, :Q, :]
